# Initial kernel scaffold; baseline (speedup 1.0000x reference)
import jax
import jax.numpy as jnp
from jax.experimental import pallas as pl


def kernel(node_attributes, edge_node_indices, edge_attributes, W_msg, b_msg, W_upd, b_upd):
    raise NotImplementedError("write your pallas kernel here")


# speedup vs baseline: 35.9148x; 35.9148x over previous
import functools

import jax
import jax.numpy as jnp
from jax import lax
from jax.experimental import pallas as pl
from jax.experimental.pallas import tpu as pltpu
from jax.experimental.pallas import tpu_sc as plsc

N_NODES = 10000
N_EDGES = 320000
D_FEAT = 128
N_PAD = 10240
NW = 32
EPW = N_EDGES // NW

@functools.cache
def _sc_degree_partials():
    mesh = plsc.VectorSubcoreMesh(core_axis_name="c", subcore_axis_name="s")

    @functools.partial(
        pl.kernel,
        out_type=jax.ShapeDtypeStruct((NW, N_PAD), jnp.float32),
        mesh=mesh,
        scratch_types=[
            pltpu.VMEM((EPW,), jnp.int32),
            pltpu.VMEM((N_PAD,), jnp.float32),
        ],
        compiler_params=pltpu.CompilerParams(needs_layout_passes=False),
    )
    def sc_kernel(idx_hbm, out_hbm, idx_v, hist_v):
        wid = lax.axis_index("s") * 2 + lax.axis_index("c")
        pltpu.sync_copy(idx_hbm.at[pl.ds(wid * EPW, EPW)], idx_v)

        zeros16 = jnp.zeros((16,), jnp.float32)

        def zero_body(i, carry):
            hist_v[pl.ds(i * 16, 16)] = zeros16
            return carry

        lax.fori_loop(0, N_PAD // 16, zero_body, 0)

        def acc_body(j, carry):
            idx = idx_v[pl.ds(j * 16, 16)]
            cnt, last = plsc.scan_count(idx)
            plsc.addupdate_scatter(hist_v, [idx],
                                   cnt.astype(jnp.float32), mask=last)
            return carry

        lax.fori_loop(0, EPW // 16, acc_body, 0)
        pltpu.sync_copy(hist_v, out_hbm.at[wid])

    return sc_kernel


ROWS = 1024
GRID = N_PAD // ROWS


def _tc_update_body(x_ref, p_ref, w1t_ref, wl_ref, b_ref, o_ref):
    deg = jnp.sum(p_ref[...], axis=0)
    xb = x_ref[...].astype(jnp.bfloat16)
    wb = w1t_ref[...].astype(jnp.bfloat16)
    u = jnp.dot(xb, wb, preferred_element_type=jnp.float32)
    degb = deg.astype(jnp.bfloat16).astype(jnp.float32)
    wlb = wl_ref[...].astype(jnp.bfloat16).astype(jnp.float32)
    u = u + degb[:, None] * wlb + b_ref[...]
    u = jnp.where(u >= 0, u, 0.01 * u)
    m = jnp.max(u, axis=1, keepdims=True)
    e = jnp.exp(u - m)
    o_ref[...] = e / jnp.sum(e, axis=1, keepdims=True)


_tc_update = pl.pallas_call(
    _tc_update_body,
    grid=(GRID,),
    in_specs=[
        pl.BlockSpec((ROWS, D_FEAT), lambda i: (i, 0)),
        pl.BlockSpec((NW, ROWS), lambda i: (0, i)),
        pl.BlockSpec((D_FEAT, D_FEAT), lambda i: (0, 0)),
        pl.BlockSpec((1, D_FEAT), lambda i: (0, 0)),
        pl.BlockSpec((1, D_FEAT), lambda i: (0, 0)),
    ],
    out_specs=pl.BlockSpec((ROWS, D_FEAT), lambda i: (i, 0)),
    out_shape=jax.ShapeDtypeStruct((N_PAD, D_FEAT), jnp.float32),
)


def kernel(node_attributes, edge_node_indices, edge_attributes, W_msg,
           b_msg, W_upd, b_upd):
    idx0 = edge_node_indices[0].astype(jnp.int32)
    partials = _sc_degree_partials()(idx0)
    x_pad = jnp.zeros((N_PAD, D_FEAT), jnp.float32).at[:N_NODES].set(
        node_attributes)
    w1t = W_upd[:, :D_FEAT].T
    wl = W_upd[:, D_FEAT:].T
    b = b_upd[None, :]
    out = _tc_update(x_pad, partials, w1t, wl, b)
    return out[:N_NODES]


# speedup vs baseline: 47.6643x; 1.3271x over previous
import functools

import jax
import jax.numpy as jnp
from jax import lax
from jax.experimental import pallas as pl
from jax.experimental.pallas import tpu as pltpu
from jax.experimental.pallas import tpu_sc as plsc

N_NODES = 10000
N_EDGES = 320000
D_FEAT = 128
N_PAD = 10240
NW = 32
EPW = N_EDGES // NW

@functools.cache
def _sc_degree_partials():
    mesh = plsc.VectorSubcoreMesh(core_axis_name="c", subcore_axis_name="s")

    @functools.partial(
        pl.kernel,
        out_type=jax.ShapeDtypeStruct((NW, N_PAD), jnp.float32),
        mesh=mesh,
        scratch_types=[
            pltpu.VMEM((EPW,), jnp.int32),
            pltpu.VMEM((N_PAD,), jnp.float32),
        ],
        name="sc_degree_bincount",
        compiler_params=pltpu.CompilerParams(needs_layout_passes=False),
    )
    def sc_kernel(idx_hbm, out_hbm, idx_v, hist_v):
        wid = lax.axis_index("s") * 2 + lax.axis_index("c")
        pltpu.sync_copy(idx_hbm.at[pl.ds(wid * EPW, EPW)], idx_v)

        zeros16 = jnp.zeros((16,), jnp.float32)

        def zero_body(i, carry):
            hist_v[pl.ds(i * 16, 16)] = zeros16
            return carry

        lax.fori_loop(0, N_PAD // 16, zero_body, 0)

        def acc_body(j, carry):
            idx = idx_v[pl.ds(j * 16, 16)]
            cnt, last = plsc.scan_count(idx)
            plsc.addupdate_scatter(hist_v, [idx],
                                   cnt.astype(jnp.float32), mask=last)
            return carry

        lax.fori_loop(0, EPW // 16, acc_body, 0)
        pltpu.sync_copy(hist_v, out_hbm.at[wid])

    return sc_kernel


ROWS = 1024
GRID = N_PAD // ROWS


def _tc_update_body(x_ref, p_ref, w1t_ref, wl_ref, b_ref, o_ref):
    deg = jnp.sum(p_ref[...], axis=0)
    xb = x_ref[...].astype(jnp.bfloat16)
    wb = w1t_ref[...].astype(jnp.bfloat16)
    u = jnp.dot(xb, wb, preferred_element_type=jnp.float32)
    degb = deg.astype(jnp.bfloat16).astype(jnp.float32)
    wlb = wl_ref[...].astype(jnp.bfloat16).astype(jnp.float32)
    u = u + degb[:, None] * wlb + b_ref[...]
    u = jnp.where(u >= 0, u, 0.01 * u)
    m = jnp.max(u, axis=1, keepdims=True)
    e = jnp.exp(u - m)
    o_ref[...] = e / jnp.sum(e, axis=1, keepdims=True)


_tc_update = pl.pallas_call(
    _tc_update_body,
    grid=(GRID,),
    in_specs=[
        pl.BlockSpec((ROWS, D_FEAT), lambda i: (i, 0)),
        pl.BlockSpec((NW, ROWS), lambda i: (0, i)),
        pl.BlockSpec((D_FEAT, D_FEAT), lambda i: (0, 0)),
        pl.BlockSpec((1, D_FEAT), lambda i: (0, 0)),
        pl.BlockSpec((1, D_FEAT), lambda i: (0, 0)),
    ],
    out_specs=pl.BlockSpec((ROWS, D_FEAT), lambda i: (i, 0)),
    out_shape=jax.ShapeDtypeStruct((N_NODES, D_FEAT), jnp.float32),
)


def kernel(node_attributes, edge_node_indices, edge_attributes, W_msg,
           b_msg, W_upd, b_upd):
    idx_flat = edge_node_indices.astype(jnp.int32).reshape(2 * N_EDGES)
    partials = _sc_degree_partials()(idx_flat)
    w1t = W_upd[:, :D_FEAT].T
    wl = W_upd[:, D_FEAT:].T
    b = b_upd[None, :]
    return _tc_update(node_attributes, partials, w1t, wl, b)


# speedup vs baseline: 61.5935x; 1.2922x over previous
import functools

import jax
import jax.numpy as jnp
from jax import lax
from jax.experimental import pallas as pl
from jax.experimental.pallas import tpu as pltpu
from jax.experimental.pallas import tpu_sc as plsc

N_NODES = 10000
N_EDGES = 320000
D_FEAT = 128
N_PAD = 10240
NW = 32
EPW = N_EDGES // NW

@functools.cache
def _sc_degree_partials():
    mesh = plsc.VectorSubcoreMesh(core_axis_name="c", subcore_axis_name="s")

    @functools.partial(
        pl.kernel,
        out_type=jax.ShapeDtypeStruct((NW, N_PAD), jnp.float32),
        mesh=mesh,
        scratch_types=[
            pltpu.VMEM((EPW,), jnp.int32),
            pltpu.VMEM((N_PAD,), jnp.float32),
        ],
        name="sc_degree_bincount",
        compiler_params=pltpu.CompilerParams(needs_layout_passes=False),
    )
    def sc_kernel(idx_hbm, out_hbm, idx_v, hist_v):
        wid = lax.axis_index("s") * 2 + lax.axis_index("c")
        pltpu.sync_copy(idx_hbm.at[pl.ds(wid * EPW, EPW)], idx_v)

        zeros16 = jnp.zeros((16,), jnp.float32)

        @plsc.parallel_loop(0, N_PAD // 16, unroll=8)
        def _(i):
            hist_v[pl.ds(i * 16, 16)] = zeros16

        @plsc.parallel_loop(0, EPW // 16, unroll=8)
        def _(j):
            idx = idx_v[pl.ds(j * 16, 16)]
            cnt, last = plsc.scan_count(idx)
            plsc.addupdate_scatter(hist_v, [idx],
                                   cnt.astype(jnp.float32), mask=last)

        pltpu.sync_copy(hist_v, out_hbm.at[wid])

    return sc_kernel


ROWS = 1024
GRID = N_PAD // ROWS


def _tc_update_body(x_ref, p_ref, w1t_ref, wl_ref, b_ref, o_ref):
    deg = jnp.sum(p_ref[...], axis=0)
    xb = x_ref[...].astype(jnp.bfloat16)
    wb = w1t_ref[...].astype(jnp.bfloat16)
    u = jnp.dot(xb, wb, preferred_element_type=jnp.float32)
    degb = deg.astype(jnp.bfloat16).astype(jnp.float32)
    wlb = wl_ref[...].astype(jnp.bfloat16).astype(jnp.float32)
    u = u + degb[:, None] * wlb + b_ref[...]
    u = jnp.where(u >= 0, u, 0.01 * u)
    m = jnp.max(u, axis=1, keepdims=True)
    e = jnp.exp(u - m)
    o_ref[...] = e / jnp.sum(e, axis=1, keepdims=True)


_tc_update = pl.pallas_call(
    _tc_update_body,
    grid=(GRID,),
    in_specs=[
        pl.BlockSpec((ROWS, D_FEAT), lambda i: (i, 0)),
        pl.BlockSpec((NW, ROWS), lambda i: (0, i)),
        pl.BlockSpec((D_FEAT, D_FEAT), lambda i: (0, 0)),
        pl.BlockSpec((1, D_FEAT), lambda i: (0, 0)),
        pl.BlockSpec((1, D_FEAT), lambda i: (0, 0)),
    ],
    out_specs=pl.BlockSpec((ROWS, D_FEAT), lambda i: (i, 0)),
    out_shape=jax.ShapeDtypeStruct((N_NODES, D_FEAT), jnp.float32),
)


def kernel(node_attributes, edge_node_indices, edge_attributes, W_msg,
           b_msg, W_upd, b_upd):
    idx_flat = edge_node_indices.astype(jnp.int32).reshape(2 * N_EDGES)
    partials = _sc_degree_partials()(idx_flat)
    w1t = W_upd[:, :D_FEAT].T
    wl = W_upd[:, D_FEAT:].T
    b = b_upd[None, :]
    return _tc_update(node_attributes, partials, w1t, wl, b)


# speedup vs baseline: 67.8511x; 1.1016x over previous
import functools

import jax
import jax.numpy as jnp
from jax import lax
from jax.experimental import pallas as pl
from jax.experimental.pallas import tpu as pltpu
from jax.experimental.pallas import tpu_sc as plsc

N_NODES = 10000
N_EDGES = 320000
D_FEAT = 128
N_PAD = 10240
NW = 32
EPW = 9984
EPW_LAST = N_EDGES - (NW - 1) * EPW
VECS = EPW // 16
VECS_LAST = EPW_LAST // 16

@functools.cache
def _sc_degree_partials():
    mesh = plsc.VectorSubcoreMesh(core_axis_name="c", subcore_axis_name="s")

    @functools.partial(
        pl.kernel,
        out_type=jax.ShapeDtypeStruct((NW, N_PAD), jnp.float32),
        mesh=mesh,
        scratch_types=[
            pltpu.VMEM((2, EPW_LAST), jnp.int32),
            pltpu.VMEM((N_PAD,), jnp.float32),
        ],
        name="sc_degree_bincount",
        compiler_params=pltpu.CompilerParams(needs_layout_passes=False),
    )
    def sc_kernel(idx_hbm, out_hbm, idx_v, hist_v):
        wid = lax.axis_index("s") * 2 + lax.axis_index("c")
        pltpu.sync_copy(idx_hbm.at[:, pl.ds(wid * EPW, EPW_LAST)], idx_v)
        limit = jnp.where(wid == NW - 1, VECS_LAST, VECS)

        zeros16 = jnp.zeros((16,), jnp.float32)

        @plsc.parallel_loop(0, N_PAD // 16, unroll=8)
        def _(i):
            hist_v[pl.ds(i * 16, 16)] = zeros16

        @plsc.parallel_loop(0, VECS_LAST, unroll=8)
        def _(j):
            idx = idx_v[0, pl.ds(j * 16, 16)]
            cnt, last = plsc.scan_count(idx)
            plsc.addupdate_scatter(hist_v, [idx],
                                   cnt.astype(jnp.float32),
                                   mask=last & (j < limit))

        pltpu.sync_copy(hist_v, out_hbm.at[wid])

    return sc_kernel


ROWS = 2048
GRID = N_PAD // ROWS


def _tc_update_body(x_ref, p_ref, w1t_ref, wl_ref, b_ref, o_ref):
    deg = jnp.sum(p_ref[...], axis=0)
    xb = x_ref[...].astype(jnp.bfloat16)
    wb = w1t_ref[...].astype(jnp.bfloat16)
    u = jnp.dot(xb, wb, preferred_element_type=jnp.float32)
    degb = deg.astype(jnp.bfloat16).astype(jnp.float32)
    wlb = wl_ref[...].astype(jnp.bfloat16).astype(jnp.float32)
    u = u + degb[:, None] * wlb + b_ref[...]
    u = jnp.where(u >= 0, u, 0.01 * u)
    m = jnp.max(u, axis=1, keepdims=True)
    e = jnp.exp(u - m)
    o_ref[...] = e / jnp.sum(e, axis=1, keepdims=True)


_tc_update = pl.pallas_call(
    _tc_update_body,
    grid=(GRID,),
    in_specs=[
        pl.BlockSpec((ROWS, D_FEAT), lambda i: (i, 0)),
        pl.BlockSpec((NW, ROWS), lambda i: (0, i)),
        pl.BlockSpec((D_FEAT, D_FEAT), lambda i: (0, 0)),
        pl.BlockSpec((1, D_FEAT), lambda i: (0, 0)),
        pl.BlockSpec((1, D_FEAT), lambda i: (0, 0)),
    ],
    out_specs=pl.BlockSpec((ROWS, D_FEAT), lambda i: (i, 0)),
    out_shape=jax.ShapeDtypeStruct((N_NODES, D_FEAT), jnp.float32),
)


def kernel(node_attributes, edge_node_indices, edge_attributes, W_msg,
           b_msg, W_upd, b_upd):
    partials = _sc_degree_partials()(edge_node_indices.astype(jnp.int32))
    w1t = W_upd[:, :D_FEAT].T
    wl = W_upd[:, D_FEAT:].T
    b = b_upd[None, :]
    return _tc_update(node_attributes, partials, w1t, wl, b)


# speedup vs baseline: 68.6440x; 1.0117x over previous
import functools

import jax
import jax.numpy as jnp
from jax import lax
from jax.experimental import pallas as pl
from jax.experimental.pallas import tpu as pltpu
from jax.experimental.pallas import tpu_sc as plsc

N_NODES = 10000
N_EDGES = 320000
D_FEAT = 128
N_PAD = 10240
NW = 32
EPW = 9984
EPW_LAST = N_EDGES - (NW - 1) * EPW
VECS = EPW // 16
VECS_LAST = EPW_LAST // 16

@functools.cache
def _sc_degree_partials():
    mesh = plsc.VectorSubcoreMesh(core_axis_name="c", subcore_axis_name="s")

    @functools.partial(
        pl.kernel,
        out_type=jax.ShapeDtypeStruct((NW, N_PAD), jnp.float32),
        mesh=mesh,
        scratch_types=[
            pltpu.VMEM((2, EPW_LAST), jnp.int32),
            pltpu.VMEM((N_PAD,), jnp.float32),
        ],
        name="sc_degree_bincount",
        compiler_params=pltpu.CompilerParams(needs_layout_passes=False),
    )
    def sc_kernel(idx_hbm, out_hbm, idx_v, hist_v):
        wid = lax.axis_index("s") * 2 + lax.axis_index("c")
        pltpu.sync_copy(idx_hbm.at[:, pl.ds(wid * EPW, EPW_LAST)], idx_v)
        limit = jnp.where(wid == NW - 1, VECS_LAST, VECS)

        zeros16 = jnp.zeros((16,), jnp.float32)

        @plsc.parallel_loop(0, N_PAD // 16, unroll=8)
        def _(i):
            hist_v[pl.ds(i * 16, 16)] = zeros16

        @plsc.parallel_loop(0, VECS_LAST, unroll=8)
        def _(j):
            idx = idx_v[0, pl.ds(j * 16, 16)]
            cnt, last = plsc.scan_count(idx)
            plsc.addupdate_scatter(hist_v, [idx],
                                   cnt.astype(jnp.float32),
                                   mask=last & (j < limit))

        pltpu.sync_copy(hist_v, out_hbm.at[wid])

    return sc_kernel


ROWS = 2048
GRID = N_PAD // ROWS


def _tc_update_body(x_ref, p_ref, w_ref, b_ref, o_ref):
    deg = jnp.sum(p_ref[...], axis=0)
    xb = x_ref[...].astype(jnp.bfloat16)
    degb = deg.astype(jnp.bfloat16)
    xcat = jnp.concatenate([xb, degb[:, None]], axis=1)
    wb = w_ref[...].astype(jnp.bfloat16)
    u = jax.lax.dot_general(xcat, wb, (((1,), (1,)), ((), ())),
                            preferred_element_type=jnp.float32)
    u = u + b_ref[...][None, :]
    u = jnp.where(u >= 0, u, 0.01 * u)
    m = jnp.max(u, axis=1, keepdims=True)
    e = jnp.exp(u - m)
    o_ref[...] = e / jnp.sum(e, axis=1, keepdims=True)


_tc_update = pl.pallas_call(
    _tc_update_body,
    grid=(GRID,),
    in_specs=[
        pl.BlockSpec((ROWS, D_FEAT), lambda i: (i, 0)),
        pl.BlockSpec((NW, ROWS), lambda i: (0, i)),
        pl.BlockSpec((D_FEAT, D_FEAT + 1), lambda i: (0, 0)),
        pl.BlockSpec((D_FEAT,), lambda i: (0,)),
    ],
    out_specs=pl.BlockSpec((ROWS, D_FEAT), lambda i: (i, 0)),
    out_shape=jax.ShapeDtypeStruct((N_NODES, D_FEAT), jnp.float32),
)


def kernel(node_attributes, edge_node_indices, edge_attributes, W_msg,
           b_msg, W_upd, b_upd):
    partials = _sc_degree_partials()(edge_node_indices.astype(jnp.int32))
    return _tc_update(node_attributes, partials, W_upd, b_upd)


# speedup vs baseline: 70.4619x; 1.0265x over previous
import functools

import jax
import jax.numpy as jnp
from jax import lax
from jax.experimental import pallas as pl
from jax.experimental.pallas import tpu as pltpu
from jax.experimental.pallas import tpu_sc as plsc

N_NODES = 10000
N_EDGES = 320000
D_FEAT = 128
N_PAD = 10240
NW = 32
EPW = 9984
EPW_LAST = N_EDGES - (NW - 1) * EPW
VECS = EPW // 16
VECS_LAST = EPW_LAST // 16

@functools.cache
def _sc_degree_partials():
    mesh = plsc.VectorSubcoreMesh(core_axis_name="c", subcore_axis_name="s")

    @functools.partial(
        pl.kernel,
        out_type=jax.ShapeDtypeStruct((NW, N_PAD), jnp.float32),
        mesh=mesh,
        scratch_types=[
            pltpu.VMEM((2, EPW_LAST), jnp.int32),
            pltpu.VMEM((N_PAD,), jnp.float32),
            pltpu.SemaphoreType.DMA,
        ],
        name="sc_degree_bincount",
        compiler_params=pltpu.CompilerParams(needs_layout_passes=False),
    )
    def sc_kernel(idx_hbm, out_hbm, idx_v, hist_v, dma_sem):
        wid = lax.axis_index("s") * 2 + lax.axis_index("c")
        cp = pltpu.async_copy(idx_hbm.at[:, pl.ds(wid * EPW, EPW_LAST)],
                              idx_v, dma_sem)
        limit = jnp.where(wid == NW - 1, VECS_LAST, VECS)

        zeros16 = jnp.zeros((16,), jnp.float32)

        @plsc.parallel_loop(0, N_PAD // 16, unroll=8)
        def _(i):
            hist_v[pl.ds(i * 16, 16)] = zeros16

        cp.wait()

        @plsc.parallel_loop(0, VECS_LAST, unroll=16)
        def _(j):
            idx = idx_v[0, pl.ds(j * 16, 16)]
            cnt, last = plsc.scan_count(idx)
            plsc.addupdate_scatter(hist_v, [idx],
                                   cnt.astype(jnp.float32),
                                   mask=last & (j < limit))

        pltpu.sync_copy(hist_v, out_hbm.at[wid])

    return sc_kernel


ROWS = 2048
GRID = N_PAD // ROWS


def _tc_update_body(x_ref, p_ref, w_ref, b_ref, o_ref):
    deg = jnp.sum(p_ref[...], axis=0)
    xb = x_ref[...].astype(jnp.bfloat16)
    degb = deg.astype(jnp.bfloat16)
    xcat = jnp.concatenate([xb, degb[:, None]], axis=1)
    wb = w_ref[...].astype(jnp.bfloat16)
    u = jax.lax.dot_general(xcat, wb, (((1,), (1,)), ((), ())),
                            preferred_element_type=jnp.float32)
    u = u + b_ref[...][None, :]
    u = jnp.where(u >= 0, u, 0.01 * u)
    m = jnp.max(u, axis=1, keepdims=True)
    e = jnp.exp(u - m)
    o_ref[...] = e / jnp.sum(e, axis=1, keepdims=True)


_tc_update = pl.pallas_call(
    _tc_update_body,
    grid=(GRID,),
    in_specs=[
        pl.BlockSpec((ROWS, D_FEAT), lambda i: (i, 0)),
        pl.BlockSpec((NW, ROWS), lambda i: (0, i)),
        pl.BlockSpec((D_FEAT, D_FEAT + 1), lambda i: (0, 0)),
        pl.BlockSpec((D_FEAT,), lambda i: (0,)),
    ],
    out_specs=pl.BlockSpec((ROWS, D_FEAT), lambda i: (i, 0)),
    out_shape=jax.ShapeDtypeStruct((N_NODES, D_FEAT), jnp.float32),
)


def kernel(node_attributes, edge_node_indices, edge_attributes, W_msg,
           b_msg, W_upd, b_upd):
    partials = _sc_degree_partials()(edge_node_indices.astype(jnp.int32))
    return _tc_update(node_attributes, partials, W_upd, b_upd)


# speedup vs baseline: 70.5538x; 1.0013x over previous
import functools

import jax
import jax.numpy as jnp
from jax import lax
from jax.experimental import pallas as pl
from jax.experimental.pallas import tpu as pltpu
from jax.experimental.pallas import tpu_sc as plsc

N_NODES = 10000
N_EDGES = 320000
D_FEAT = 128
N_PAD = 10240
NW = 32
EPW = 9984
EPW_LAST = N_EDGES - (NW - 1) * EPW
VECS = EPW // 16
VECS_LAST = EPW_LAST // 16

@functools.cache
def _sc_degree_partials():
    mesh = plsc.VectorSubcoreMesh(core_axis_name="c", subcore_axis_name="s")

    @functools.partial(
        pl.kernel,
        out_type=jax.ShapeDtypeStruct((NW, N_PAD), jnp.float32),
        mesh=mesh,
        scratch_types=[
            pltpu.VMEM((2, EPW_LAST), jnp.int32),
            pltpu.VMEM((N_PAD,), jnp.float32),
            pltpu.SemaphoreType.DMA,
        ],
        name="sc_degree_bincount",
        compiler_params=pltpu.CompilerParams(needs_layout_passes=False),
    )
    def sc_kernel(idx_hbm, out_hbm, idx_v, hist_v, dma_sem):
        wid = lax.axis_index("s") * 2 + lax.axis_index("c")
        cp = pltpu.async_copy(idx_hbm.at[:, pl.ds(wid * EPW, EPW_LAST)],
                              idx_v, dma_sem)
        limit = jnp.where(wid == NW - 1, VECS_LAST, VECS)

        zeros16 = jnp.zeros((16,), jnp.float32)

        @plsc.parallel_loop(0, N_PAD // 16, unroll=8)
        def _(i):
            hist_v[pl.ds(i * 16, 16)] = zeros16

        cp.wait()

        @plsc.parallel_loop(0, VECS_LAST, unroll=16)
        def _(j):
            idx = idx_v[0, pl.ds(j * 16, 16)]
            cnt, last = plsc.scan_count(idx)
            plsc.addupdate_scatter(hist_v, [idx],
                                   cnt.astype(jnp.float32),
                                   mask=last & (j < limit))

        pltpu.sync_copy(hist_v, out_hbm.at[wid])

    return sc_kernel


ROWS = 2048
GRID = N_PAD // ROWS


def _tc_update_body(x_ref, p_ref, w_ref, b_ref, o_ref):
    deg = jnp.sum(p_ref[...], axis=0)
    xb = x_ref[...].astype(jnp.bfloat16)
    degb = deg.astype(jnp.bfloat16)
    xcat = jnp.concatenate([xb, degb[:, None]], axis=1)
    wb = w_ref[...].astype(jnp.bfloat16)
    u = jax.lax.dot_general(xcat, wb, (((1,), (1,)), ((), ())),
                            preferred_element_type=jnp.float32)
    u = u + b_ref[...][None, :]
    u = jnp.where(u >= 0, u, 0.01 * u)
    m = jnp.max(u, axis=1, keepdims=True)
    e = jnp.exp(u - m)
    o_ref[...] = e / jnp.sum(e, axis=1, keepdims=True)


_tc_update = pl.pallas_call(
    _tc_update_body,
    grid=(GRID,),
    in_specs=[
        pl.BlockSpec((ROWS, D_FEAT), lambda i: (i, 0)),
        pl.BlockSpec((NW, ROWS), lambda i: (0, i)),
        pl.BlockSpec((D_FEAT, D_FEAT + 1), lambda i: (0, 0)),
        pl.BlockSpec((D_FEAT,), lambda i: (0,)),
    ],
    out_specs=pl.BlockSpec((ROWS, D_FEAT), lambda i: (i, 0)),
    out_shape=jax.ShapeDtypeStruct((N_NODES, D_FEAT), jnp.float32),
    compiler_params=pltpu.CompilerParams(dimension_semantics=("parallel",)),
)


def kernel(node_attributes, edge_node_indices, edge_attributes, W_msg,
           b_msg, W_upd, b_upd):
    partials = _sc_degree_partials()(edge_node_indices.astype(jnp.int32))
    return _tc_update(node_attributes, partials, W_upd, b_upd)


# speedup vs baseline: 72.4848x; 1.0274x over previous
import functools

import jax
import jax.numpy as jnp
from jax import lax
from jax.experimental import pallas as pl
from jax.experimental.pallas import tpu as pltpu
from jax.experimental.pallas import tpu_sc as plsc

N_NODES = 10000
N_EDGES = 320000
D_FEAT = 128
N_PAD = 10240
NW = 32
EPW = 9984
EPW_LAST = N_EDGES - (NW - 1) * EPW
VECS = EPW // 16
VECS_LAST = EPW_LAST // 16

@functools.cache
def _sc_degree_partials():
    mesh = plsc.VectorSubcoreMesh(core_axis_name="c", subcore_axis_name="s")

    @functools.partial(
        pl.kernel,
        out_type=jax.ShapeDtypeStruct((NW, N_PAD), jnp.float32),
        mesh=mesh,
        scratch_types=[
            pltpu.VMEM((2, EPW_LAST), jnp.int32),
            pltpu.VMEM((N_PAD,), jnp.float32),
            pltpu.SemaphoreType.DMA,
        ],
        name="sc_degree_bincount",
        compiler_params=pltpu.CompilerParams(needs_layout_passes=False),
    )
    def sc_kernel(idx_hbm, out_hbm, idx_v, hist_v, dma_sem):
        wid = lax.axis_index("s") * 2 + lax.axis_index("c")
        cp = pltpu.async_copy(idx_hbm.at[:, pl.ds(wid * EPW, EPW_LAST)],
                              idx_v, dma_sem)
        limit = jnp.where(wid == NW - 1, VECS_LAST, VECS)

        zeros16 = jnp.zeros((16,), jnp.float32)

        @plsc.parallel_loop(0, N_PAD // 16, unroll=8)
        def _(i):
            hist_v[pl.ds(i * 16, 16)] = zeros16

        cp.wait()

        @plsc.parallel_loop(0, VECS_LAST, unroll=16)
        def _(j):
            idx = idx_v[0, pl.ds(j * 16, 16)]
            cnt, last = plsc.scan_count(idx)
            plsc.addupdate_scatter(hist_v, [idx],
                                   cnt.astype(jnp.float32),
                                   mask=last & (j < limit))

        pltpu.sync_copy(hist_v, out_hbm.at[wid])

    return sc_kernel


ROWS = 2560
GRID = N_PAD // ROWS


def _tc_update_body(x_ref, p_ref, w_ref, b_ref, o_ref):
    deg = jnp.sum(p_ref[...], axis=0)
    xb = x_ref[...].astype(jnp.bfloat16)
    degb = deg.astype(jnp.bfloat16)
    xcat = jnp.concatenate([xb, degb[:, None]], axis=1)
    wb = w_ref[...].astype(jnp.bfloat16)
    u = jax.lax.dot_general(xcat, wb, (((1,), (1,)), ((), ())),
                            preferred_element_type=jnp.float32)
    u = u + b_ref[...][None, :]
    u = jnp.where(u >= 0, u, 0.01 * u)
    m = jnp.max(u, axis=1, keepdims=True)
    e = jnp.exp(u - m)
    o_ref[...] = e / jnp.sum(e, axis=1, keepdims=True)


_tc_update = pl.pallas_call(
    _tc_update_body,
    grid=(GRID,),
    in_specs=[
        pl.BlockSpec((ROWS, D_FEAT), lambda i: (i, 0)),
        pl.BlockSpec((NW, ROWS), lambda i: (0, i)),
        pl.BlockSpec((D_FEAT, D_FEAT + 1), lambda i: (0, 0)),
        pl.BlockSpec((D_FEAT,), lambda i: (0,)),
    ],
    out_specs=pl.BlockSpec((ROWS, D_FEAT), lambda i: (i, 0)),
    out_shape=jax.ShapeDtypeStruct((N_NODES, D_FEAT), jnp.float32),
    compiler_params=pltpu.CompilerParams(dimension_semantics=("parallel",)),
)


def kernel(node_attributes, edge_node_indices, edge_attributes, W_msg,
           b_msg, W_upd, b_upd):
    partials = _sc_degree_partials()(edge_node_indices.astype(jnp.int32))
    return _tc_update(node_attributes, partials, W_upd, b_upd)


# speedup vs baseline: 74.3491x; 1.0257x over previous
import functools

import jax
import jax.numpy as jnp
from jax import lax
from jax.experimental import pallas as pl
from jax.experimental.pallas import tpu as pltpu
from jax.experimental.pallas import tpu_sc as plsc

N_NODES = 10000
N_EDGES = 320000
D_FEAT = 128
N_PAD = 10240
NW = 32
EPW = 9984
EPW_LAST = N_EDGES - (NW - 1) * EPW
VECS = EPW // 16
VECS_LAST = EPW_LAST // 16

@functools.cache
def _sc_degree_partials():
    mesh = plsc.VectorSubcoreMesh(core_axis_name="c", subcore_axis_name="s")

    @functools.partial(
        pl.kernel,
        out_type=jax.ShapeDtypeStruct((NW, N_PAD), jnp.float32),
        mesh=mesh,
        scratch_types=[
            pltpu.VMEM((2, EPW_LAST), jnp.int32),
            pltpu.VMEM((N_PAD,), jnp.float32),
            pltpu.SemaphoreType.DMA,
        ],
        name="sc_degree_bincount",
        compiler_params=pltpu.CompilerParams(needs_layout_passes=False),
    )
    def sc_kernel(idx_hbm, out_hbm, idx_v, hist_v, dma_sem):
        wid = lax.axis_index("s") * 2 + lax.axis_index("c")
        cp = pltpu.async_copy(idx_hbm.at[:, pl.ds(wid * EPW, EPW_LAST)],
                              idx_v, dma_sem)
        limit = jnp.where(wid == NW - 1, VECS_LAST, VECS)

        zeros16 = jnp.zeros((16,), jnp.float32)

        @plsc.parallel_loop(0, N_PAD // 16, unroll=8)
        def _(i):
            hist_v[pl.ds(i * 16, 16)] = zeros16

        cp.wait()

        @plsc.parallel_loop(0, VECS_LAST, unroll=16)
        def _(j):
            idx = idx_v[0, pl.ds(j * 16, 16)]
            cnt, last = plsc.scan_count(idx)
            plsc.addupdate_scatter(hist_v, [idx],
                                   cnt.astype(jnp.float32),
                                   mask=last & (j < limit))

        pltpu.sync_copy(hist_v, out_hbm.at[wid])

    return sc_kernel


ROWS = 5120
GRID = N_PAD // ROWS


def _tc_update_body(x_ref, p_ref, w_ref, b_ref, o_ref):
    deg = jnp.sum(p_ref[...], axis=0)
    xb = x_ref[...].astype(jnp.bfloat16)
    degb = deg.astype(jnp.bfloat16)
    xcat = jnp.concatenate([xb, degb[:, None]], axis=1)
    wb = w_ref[...].astype(jnp.bfloat16)
    u = jax.lax.dot_general(xcat, wb, (((1,), (1,)), ((), ())),
                            preferred_element_type=jnp.float32)
    u = u + b_ref[...][None, :]
    u = jnp.where(u >= 0, u, 0.01 * u)
    m = jnp.max(u, axis=1, keepdims=True)
    e = jnp.exp(u - m)
    o_ref[...] = e / jnp.sum(e, axis=1, keepdims=True)


_tc_update = pl.pallas_call(
    _tc_update_body,
    grid=(GRID,),
    in_specs=[
        pl.BlockSpec((ROWS, D_FEAT), lambda i: (i, 0)),
        pl.BlockSpec((NW, ROWS), lambda i: (0, i)),
        pl.BlockSpec((D_FEAT, D_FEAT + 1), lambda i: (0, 0)),
        pl.BlockSpec((D_FEAT,), lambda i: (0,)),
    ],
    out_specs=pl.BlockSpec((ROWS, D_FEAT), lambda i: (i, 0)),
    out_shape=jax.ShapeDtypeStruct((N_NODES, D_FEAT), jnp.float32),
    compiler_params=pltpu.CompilerParams(dimension_semantics=("parallel",)),
)


def kernel(node_attributes, edge_node_indices, edge_attributes, W_msg,
           b_msg, W_upd, b_upd):
    partials = _sc_degree_partials()(edge_node_indices.astype(jnp.int32))
    return _tc_update(node_attributes, partials, W_upd, b_upd)
